# baseline (speedup 1.0000x reference)
import functools

import jax
import jax.numpy as jnp
from jax import lax
from jax.experimental import pallas as pl
from jax.experimental.pallas import tpu as pltpu
from jax.experimental.pallas import tpu_sc as plsc

NC = 2
NS = 16
NW = NC * NS

V = 100
D = 3
VPAD = 128
SLAB = 512
NB = 4


def _sc_lookup(ids_t, table_cols):
    t_dim, b_dim = ids_t.shape
    wcols = b_dim // NW
    n_slabs = t_dim // 8

    mesh = plsc.VectorSubcoreMesh(core_axis_name="c", subcore_axis_name="s")

    scratch = [pltpu.VMEM((VPAD,), jnp.float32)] * D
    for _ in range(NB):
        scratch.append(pltpu.VMEM((8, SLAB), jnp.int32))
        scratch.extend([pltpu.VMEM((8, SLAB), jnp.float32)] * D)
    scratch.extend([pltpu.SemaphoreType.DMA] * (2 * NB))

    @functools.partial(
        pl.kernel,
        mesh=mesh,
        out_type=jax.ShapeDtypeStruct((D, t_dim, b_dim), jnp.float32),
        scratch_types=scratch,
        compiler_params=pltpu.CompilerParams(needs_layout_passes=False),
    )
    def k(tc0_hbm, tc1_hbm, tc2_hbm, ids_hbm, out_hbm, *refs):
        tcols = refs[:D]
        banks = []
        for i in range(NB):
            base = D + 4 * i
            banks.append((refs[base], refs[base + 1 : base + 4]))
        sems = refs[D + 4 * NB :]
        semi = sems[:NB]
        semo = sems[NB:]

        wid = lax.axis_index("s") * NC + lax.axis_index("c")
        for d, src in enumerate((tc0_hbm, tc1_hbm, tc2_hbm)):
            pltpu.sync_copy(src, tcols[d])
        cbase = wid * wcols

        def ids_src(s):
            return ids_hbm.at[pl.ds(s * 8, 8), pl.ds(cbase, SLAB)]

        def out_dst(s, d):
            return out_hbm.at[d, pl.ds(s * 8, 8), pl.ds(cbase, SLAB)]

        def start_ids(s, b):
            pltpu.make_async_copy(ids_src(s), banks[b][0], semi[b]).start()

        def wait_ids(s, b):
            pltpu.make_async_copy(ids_src(s), banks[b][0], semi[b]).wait()

        def start_outs(s, b):
            for d in range(D):
                pltpu.make_async_copy(banks[b][1][d], out_dst(s, d), semo[b]).start()

        def wait_outs(s, b):
            for d in range(D):
                pltpu.make_async_copy(banks[b][1][d], out_dst(s, d), semo[b]).wait()

        def compute(b):
            ids_v = banks[b][0]
            outs = banks[b][1]
            for r in range(8):

                @plsc.parallel_loop(0, SLAB // 16, unroll=4)
                def _(g, r=r):
                    c = g * 16
                    ids16 = ids_v[r, pl.ds(c, 16)]
                    for d in range(D):
                        outs[d][r, pl.ds(c, 16)] = plsc.load_gather(
                            tcols[d], [ids16]
                        )

        def process(s, b):
            wait_ids(s, b)

            @pl.when(s >= NB)
            def _():
                wait_outs(s - NB, b)

            compute(b)
            start_outs(s, b)

            @pl.when(s + NB < n_slabs)
            def _():
                start_ids(s + NB, b)

        for b in range(NB):
            start_ids(b, b)

        def body(j, carry):
            for b in range(NB):
                process(NB * j + b, b)
            return carry

        lax.fori_loop(0, n_slabs // NB, body, 0)
        for s in range(NB * (n_slabs // NB), n_slabs):
            process(s, s % NB)
        for s in range(n_slabs - NB, n_slabs):
            wait_outs(s, s % NB)

    return k(table_cols[0], table_cols[1], table_cols[2], ids_t)


def kernel(entity_ids, entity_table):
    ids_t = entity_ids.T.astype(jnp.int32)
    tc = jnp.zeros((D, VPAD), jnp.float32).at[:, :V].set(entity_table.T)
    table_cols = (tc[0], tc[1], tc[2])
    out_t = _sc_lookup(ids_t, table_cols)
    return out_t.transpose(2, 1, 0)


# speedup vs baseline: 1.0914x; 1.0914x over previous
import functools

import jax
import jax.numpy as jnp
from jax import lax
from jax.experimental import pallas as pl
from jax.experimental.pallas import tpu as pltpu
from jax.experimental.pallas import tpu_sc as plsc

NC = 2
NS = 16
NW = NC * NS

V = 100
D = 3
VPAD = 128
SLAB = 512


def _sc_lookup(ids_t, table_cols):
    t_dim, b_dim = ids_t.shape
    wcols = b_dim // NW
    n_slabs = t_dim // 8

    mesh = plsc.VectorSubcoreMesh(core_axis_name="c", subcore_axis_name="s")

    @functools.partial(
        pl.kernel,
        mesh=mesh,
        out_type=jax.ShapeDtypeStruct((D, t_dim, b_dim), jnp.float32),
        scratch_types=[
            pltpu.VMEM((VPAD,), jnp.float32),
            pltpu.VMEM((VPAD,), jnp.float32),
            pltpu.VMEM((VPAD,), jnp.float32),
            pltpu.VMEM((8, SLAB), jnp.int32),
            pltpu.VMEM((8, SLAB), jnp.int32),
            pltpu.VMEM((8, SLAB), jnp.float32),
            pltpu.VMEM((8, SLAB), jnp.float32),
            pltpu.VMEM((8, SLAB), jnp.float32),
            pltpu.VMEM((8, SLAB), jnp.float32),
            pltpu.VMEM((8, SLAB), jnp.float32),
            pltpu.VMEM((8, SLAB), jnp.float32),
            pltpu.SemaphoreType.DMA,
            pltpu.SemaphoreType.DMA,
            pltpu.SemaphoreType.DMA,
            pltpu.SemaphoreType.DMA,
        ],
        compiler_params=pltpu.CompilerParams(needs_layout_passes=False),
    )
    def k(tc0_hbm, tc1_hbm, tc2_hbm, ids_hbm, out_hbm, t0, t1, t2,
          ids_a, ids_b, oa0, oa1, oa2, ob0, ob1, ob2,
          semi_a, semi_b, semo_a, semo_b):
        wid = lax.axis_index("s") * NC + lax.axis_index("c")
        tcols = (t0, t1, t2)
        for d, src in enumerate((tc0_hbm, tc1_hbm, tc2_hbm)):
            pltpu.sync_copy(src, tcols[d])
        cbase = wid * wcols
        banks = (
            (ids_a, (oa0, oa1, oa2), semi_a, semo_a),
            (ids_b, (ob0, ob1, ob2), semi_b, semo_b),
        )

        def ids_src(s):
            return ids_hbm.at[pl.ds(s * 8, 8), pl.ds(cbase, SLAB)]

        def out_dst(s, d):
            return out_hbm.at[d, pl.ds(s * 8, 8), pl.ds(cbase, SLAB)]

        def start_ids(s, bank):
            pltpu.make_async_copy(ids_src(s), bank[0], bank[2]).start()

        def wait_ids(s, bank):
            pltpu.make_async_copy(ids_src(s), bank[0], bank[2]).wait()

        def start_outs(s, bank):
            for d in range(D):
                pltpu.make_async_copy(bank[1][d], out_dst(s, d), bank[3]).start()

        def wait_outs(s, bank):
            for d in range(D):
                pltpu.make_async_copy(bank[1][d], out_dst(s, d), bank[3]).wait()

        def compute(bank):
            ids_v = bank[0]
            outs = bank[1]
            for r in range(8):

                @plsc.parallel_loop(0, SLAB // 16, unroll=8)
                def _(g, r=r):
                    c = g * 16
                    ids16 = ids_v[r, pl.ds(c, 16)]
                    for d in range(D):
                        outs[d][r, pl.ds(c, 16)] = plsc.load_gather(
                            tcols[d], [ids16]
                        )

        def process(s, bank):
            wait_ids(s, bank)

            @pl.when(s >= 2)
            def _():
                wait_outs(s - 2, bank)

            compute(bank)
            start_outs(s, bank)

            @pl.when(s + 2 < n_slabs)
            def _():
                start_ids(s + 2, bank)

        start_ids(0, banks[0])
        start_ids(1, banks[1])

        def body(j, carry):
            process(2 * j, banks[0])
            process(2 * j + 1, banks[1])
            return carry

        lax.fori_loop(0, n_slabs // 2, body, 0)
        process(n_slabs - 1, banks[0])
        wait_outs(n_slabs - 2, banks[1])
        wait_outs(n_slabs - 1, banks[0])

    return k(table_cols[0], table_cols[1], table_cols[2], ids_t)


def kernel(entity_ids, entity_table):
    ids_t = entity_ids.T.astype(jnp.int32)
    tc = jnp.zeros((D, VPAD), jnp.float32).at[:, :V].set(entity_table.T)
    table_cols = (tc[0], tc[1], tc[2])
    out_t = _sc_lookup(ids_t, table_cols)
    return out_t.transpose(2, 1, 0)
